# Initial kernel scaffold; baseline (speedup 1.0000x reference)
import jax
import jax.numpy as jnp
from jax.experimental import pallas as pl


def kernel(features, W1, b1, W2, b2):
    raise NotImplementedError("write your pallas kernel here")


# speedup vs baseline: 27.1714x; 27.1714x over previous
import functools

import jax
import jax.numpy as jnp
from jax.experimental import pallas as pl

N = 4096
D = 128
K_NEIGHBORS = 8
BLOCK_R = 512


def _density_block_kernel(x_ref, feats_ref, o_ref):
    x = x_ref[...]
    f = feats_ref[...]
    sq_r = jnp.sum(x * x, axis=1, keepdims=True)
    sq_c = jnp.sum(f * f, axis=1)[None, :]
    g = jax.lax.dot_general(
        x, f, (((1,), (1,)), ((), ())),
        preferred_element_type=jnp.float32,
    )
    d2 = jnp.maximum(sq_r + sq_c - 2.0 * g, 0.0)

    t = jnp.full((BLOCK_R, 1), -1.0, dtype=jnp.float32)
    c = jnp.zeros((BLOCK_R, 1), dtype=jnp.int32)
    for _ in range(K_NEIGHBORS + 1):
        active = c <= K_NEIGHBORS
        masked = jnp.where(d2 > t, d2, jnp.inf)
        m = jnp.min(masked, axis=1, keepdims=True)
        cnt = jnp.sum((d2 == m).astype(jnp.int32), axis=1, keepdims=True)
        t = jnp.where(active, m, t)
        c = jnp.where(active, c + cnt, c)

    kth = jnp.sqrt(t)
    o_ref[...] = 1.0 / (kth + 1e-8)


@functools.partial(jax.jit, static_argnames=())
def _density(features):
    grid = (N // BLOCK_R,)
    out = pl.pallas_call(
        _density_block_kernel,
        grid=grid,
        in_specs=[
            pl.BlockSpec((BLOCK_R, D), lambda i: (i, 0)),
            pl.BlockSpec((N, D), lambda i: (0, 0)),
        ],
        out_specs=pl.BlockSpec((BLOCK_R, 1), lambda i: (i, 0)),
        out_shape=jax.ShapeDtypeStruct((N, 1), jnp.float32),
    )(features, features)
    return out


def kernel(features, W1, b1, W2, b2):
    return jax.lax.stop_gradient(_density(features))


# speedup vs baseline: 54.9436x; 2.0221x over previous
import functools

import jax
import jax.numpy as jnp
from jax.experimental import pallas as pl

N = 4096
D = 128
K_NEIGHBORS = 8
BLOCK_R = 512
LANES = 128
CHUNKS = N // LANES

_SORT8 = [
    (0, 1), (2, 3), (0, 2), (1, 3), (1, 2),
    (4, 5), (6, 7), (4, 6), (5, 7), (5, 6),
    (0, 4), (2, 6), (2, 4), (1, 5), (3, 7), (3, 5),
    (1, 2), (3, 4), (5, 6),
]


def _ce(a, b):
    return jnp.minimum(a, b), jnp.maximum(a, b)


def _sort8(v):
    v = list(v)
    for i, j in _SORT8:
        v[i], v[j] = _ce(v[i], v[j])
    return v


def _bitonic_merge_asc(v):
    n = len(v)
    if n == 1:
        return v
    h = n // 2
    lo, hi = [], []
    for i in range(h):
        a, b = _ce(v[i], v[i + h])
        lo.append(a)
        hi.append(b)
    return _bitonic_merge_asc(lo) + _bitonic_merge_asc(hi)


def _merge88_to9(a, b):
    x = list(a) + list(b[::-1])
    lo, hi = [], []
    for i in range(8):
        u, w = _ce(x[i], x[i + 8])
        lo.append(u)
        hi.append(w)
    e9 = hi[0]
    for h in hi[1:]:
        e9 = jnp.minimum(e9, h)
    return _bitonic_merge_asc(lo) + [e9]


def _low9of18(a, b):
    x = list(a) + list(b[::-1])
    return [jnp.minimum(x[i], x[i + 9]) for i in range(9)]


def _density_block_kernel(x_ref, ft_ref, o_ref):
    x = x_ref[...]
    ft = ft_ref[...]
    sq_r = jnp.sum(x * x, axis=1, keepdims=True)
    sq_c = jnp.sum(ft * ft, axis=0, keepdims=True)
    g2 = jax.lax.dot_general(
        x * -2.0, ft, (((1,), (0,)), ((), ())),
        preferred_element_type=jnp.float32,
    )
    e = g2 + sq_c

    cols = [e[:, i * LANES:(i + 1) * LANES] for i in range(CHUNKS)]
    s8 = [_sort8(cols[g * 8:(g + 1) * 8]) for g in range(4)]
    m1 = _merge88_to9(s8[0], s8[1])
    m2 = _merge88_to9(s8[2], s8[3])
    nine = _low9of18(m1, m2)
    cand = jnp.concatenate(nine, axis=1)

    t = jnp.full((BLOCK_R, 1), -jnp.inf, dtype=jnp.float32)
    c = jnp.zeros((BLOCK_R, 1), dtype=jnp.float32)
    for _ in range(K_NEIGHBORS + 1):
        active = c <= float(K_NEIGHBORS)
        masked = jnp.where(cand > t, cand, jnp.inf)
        m = jnp.min(masked, axis=1, keepdims=True)
        cnt = jnp.sum(jnp.where(cand == m, 1.0, 0.0), axis=1, keepdims=True)
        t = jnp.where(active, m, t)
        c = jnp.where(active, c + cnt, c)

    kth_d2 = jnp.maximum(sq_r + t, 0.0)
    o_ref[...] = 1.0 / (jnp.sqrt(kth_d2) + 1e-8)


@functools.partial(jax.jit, static_argnames=())
def _density(features):
    ft = features.T
    grid = (N // BLOCK_R,)
    out = pl.pallas_call(
        _density_block_kernel,
        grid=grid,
        in_specs=[
            pl.BlockSpec((BLOCK_R, D), lambda i: (i, 0)),
            pl.BlockSpec((D, N), lambda i: (0, 0)),
        ],
        out_specs=pl.BlockSpec((BLOCK_R, 1), lambda i: (i, 0)),
        out_shape=jax.ShapeDtypeStruct((N, 1), jnp.float32),
    )(features, ft)
    return out


def kernel(features, W1, b1, W2, b2):
    return jax.lax.stop_gradient(_density(features))


# speedup vs baseline: 56.3416x; 1.0254x over previous
import functools

import jax
import jax.numpy as jnp
from jax.experimental import pallas as pl

N = 4096
D = 128
K_NEIGHBORS = 8
BLOCK_R = 512
LANES = 128
CHUNKS = N // LANES

_SORT8 = [
    (0, 1), (2, 3), (0, 2), (1, 3), (1, 2),
    (4, 5), (6, 7), (4, 6), (5, 7), (5, 6),
    (0, 4), (2, 6), (2, 4), (1, 5), (3, 7), (3, 5),
    (1, 2), (3, 4), (5, 6),
]


def _ce(a, b):
    return jnp.minimum(a, b), jnp.maximum(a, b)


def _sort8(v):
    v = list(v)
    for i, j in _SORT8:
        v[i], v[j] = _ce(v[i], v[j])
    return v


def _bitonic_merge_asc(v):
    n = len(v)
    if n == 1:
        return v
    h = n // 2
    lo, hi = [], []
    for i in range(h):
        a, b = _ce(v[i], v[i + h])
        lo.append(a)
        hi.append(b)
    return _bitonic_merge_asc(lo) + _bitonic_merge_asc(hi)


def _merge88_to9(a, b):
    x = list(a) + list(b[::-1])
    lo, hi = [], []
    for i in range(8):
        u, w = _ce(x[i], x[i + 8])
        lo.append(u)
        hi.append(w)
    e9 = hi[0]
    for h in hi[1:]:
        e9 = jnp.minimum(e9, h)
    return _bitonic_merge_asc(lo) + [e9]


def _low9of18(a, b):
    x = list(a) + list(b[::-1])
    return [jnp.minimum(x[i], x[i + 9]) for i in range(9)]


def _density_block_kernel(x_ref, ft_ref, o_ref):
    x = x_ref[...]
    ft = ft_ref[...]
    sq_r = jnp.sum(x * x, axis=1, keepdims=True)
    sq_c = jnp.sum(ft * ft, axis=0, keepdims=True)
    g2 = jax.lax.dot_general(
        x * -2.0, ft, (((1,), (0,)), ((), ())),
        preferred_element_type=jnp.float32,
    )
    e = g2 + sq_c

    cols = [e[:, i * LANES:(i + 1) * LANES] for i in range(CHUNKS)]
    s8 = [_sort8(cols[g * 8:(g + 1) * 8]) for g in range(4)]
    m1 = _merge88_to9(s8[0], s8[1])
    m2 = _merge88_to9(s8[2], s8[3])
    nine = _low9of18(m1, m2)
    cand = jnp.concatenate(nine, axis=1)

    lane = jax.lax.broadcasted_iota(jnp.int32, (BLOCK_R, 9 * LANES), 1)
    bits = jax.lax.bitcast_convert_type(cand, jnp.int32)
    key = jnp.where(bits < 0, jnp.int32(-2147483648) - bits, bits)
    keyed = (key & jnp.int32(~2047)) | (lane & jnp.int32(2047))
    for r in range(K_NEIGHBORS + 1):
        m = jnp.min(keyed, axis=1, keepdims=True)
        if r < K_NEIGHBORS:
            keyed = jnp.where(keyed == m, jnp.int32(2147483647), keyed)
    sel = jnp.max(jnp.where(keyed == m, cand, -jnp.inf), axis=1, keepdims=True)

    kth_d2 = jnp.maximum(sq_r + sel, 0.0)
    o_ref[...] = 1.0 / (jnp.sqrt(kth_d2) + 1e-8)


@functools.partial(jax.jit, static_argnames=())
def _density(features):
    ft = features.T
    grid = (N // BLOCK_R,)
    out = pl.pallas_call(
        _density_block_kernel,
        grid=grid,
        in_specs=[
            pl.BlockSpec((BLOCK_R, D), lambda i: (i, 0)),
            pl.BlockSpec((D, N), lambda i: (0, 0)),
        ],
        out_specs=pl.BlockSpec((BLOCK_R, 1), lambda i: (i, 0)),
        out_shape=jax.ShapeDtypeStruct((N, 1), jnp.float32),
    )(features, ft)
    return out


def kernel(features, W1, b1, W2, b2):
    return jax.lax.stop_gradient(_density(features))


# speedup vs baseline: 67.3135x; 1.1947x over previous
import functools

import jax
import jax.numpy as jnp
from jax.experimental import pallas as pl

N = 4096
D = 128
K_NEIGHBORS = 8
BLOCK_R = 512
LANES = 128
CHUNKS = N // LANES

_SORT8 = [
    (0, 1), (2, 3), (0, 2), (1, 3), (1, 2),
    (4, 5), (6, 7), (4, 6), (5, 7), (5, 6),
    (0, 4), (2, 6), (2, 4), (1, 5), (3, 7), (3, 5),
    (1, 2), (3, 4), (5, 6),
]


def _ce(a, b):
    return jnp.minimum(a, b), jnp.maximum(a, b)


def _sort8(v):
    v = list(v)
    for i, j in _SORT8:
        v[i], v[j] = _ce(v[i], v[j])
    return v


def _bitonic_merge_asc(v):
    n = len(v)
    if n == 1:
        return v
    h = n // 2
    lo, hi = [], []
    for i in range(h):
        a, b = _ce(v[i], v[i + h])
        lo.append(a)
        hi.append(b)
    return _bitonic_merge_asc(lo) + _bitonic_merge_asc(hi)


def _merge88_to9(a, b):
    x = list(a) + list(b[::-1])
    lo, hi = [], []
    for i in range(8):
        u, w = _ce(x[i], x[i + 8])
        lo.append(u)
        hi.append(w)
    e9 = hi[0]
    for h in hi[1:]:
        e9 = jnp.minimum(e9, h)
    return _bitonic_merge_asc(lo) + [e9]


def _density_block_kernel(x_ref, ft_ref, o_ref):
    x = x_ref[...]
    ft = ft_ref[...]
    sq_r = jnp.sum(x * x, axis=1, keepdims=True)
    sq_c = jnp.sum(ft * ft, axis=0, keepdims=True)
    g2 = jax.lax.dot_general(
        x * -2.0, ft, (((1,), (0,)), ((), ())),
        preferred_element_type=jnp.float32,
    )
    e = g2 + sq_c

    cols = [e[:, i * LANES:(i + 1) * LANES] for i in range(CHUNKS)]
    s8 = [_sort8(cols[g * 8:(g + 1) * 8]) for g in range(4)]
    s1 = _merge88_to9(s8[0], s8[1])
    s2 = _merge88_to9(s8[2], s8[3])

    t = jnp.full((BLOCK_R, 1), -jnp.inf, dtype=jnp.float32)
    c = jnp.zeros((BLOCK_R, 1), dtype=jnp.float32)
    for r in range(K_NEIGHBORS + 1):
        m = jnp.minimum(jnp.min(s1[0], axis=1, keepdims=True),
                        jnp.min(s2[0], axis=1, keepdims=True))
        t = jnp.where(c <= float(K_NEIGHBORS), m, t)
        adv1 = s1[0] == m
        adv2 = s2[0] == m
        c = c + jnp.sum(jnp.where(adv1, 1.0, 0.0) + jnp.where(adv2, 1.0, 0.0),
                        axis=1, keepdims=True)
        for i in range(K_NEIGHBORS - r):
            s1[i] = jnp.where(adv1, s1[i + 1], s1[i])
            s2[i] = jnp.where(adv2, s2[i + 1], s2[i])

    kth_d2 = jnp.maximum(sq_r + t, 0.0)
    o_ref[...] = 1.0 / (jnp.sqrt(kth_d2) + 1e-8)


@functools.partial(jax.jit, static_argnames=())
def _density(features):
    ft = features.T
    grid = (N // BLOCK_R,)
    out = pl.pallas_call(
        _density_block_kernel,
        grid=grid,
        in_specs=[
            pl.BlockSpec((BLOCK_R, D), lambda i: (i, 0)),
            pl.BlockSpec((D, N), lambda i: (0, 0)),
        ],
        out_specs=pl.BlockSpec((BLOCK_R, 1), lambda i: (i, 0)),
        out_shape=jax.ShapeDtypeStruct((N, 1), jnp.float32),
    )(features, ft)
    return out


def kernel(features, W1, b1, W2, b2):
    return jax.lax.stop_gradient(_density(features))
